# baseline (speedup 1.0000x reference)
import functools

import jax
import jax.numpy as jnp
from jax import lax
from jax.experimental import pallas as pl
from jax.experimental.pallas import tpu as pltpu
from jax.experimental.pallas import tpu_sc as plsc

N = 10000
NP = 10240
D = 256
CHUNK = 128
NC = 2
NS = 16
ROW_BLK = 400


def _dis_from_partials(degp):
    deg = degp[0, :, 0] + degp[1, :, 0] + 1.0
    return jnp.where(deg > 0.0, lax.rsqrt(jnp.maximum(deg, 1e-12)), 0.0)


def _make_sc_deg(rows):
    rows_sc = rows // NC
    rows_t = rows_sc // NS
    nfl = NP // NS
    mesh = plsc.VectorSubcoreMesh(core_axis_name="c", subcore_axis_name="s")

    @functools.partial(
        pl.kernel,
        out_type=jax.ShapeDtypeStruct((NC, NP, 128), jnp.float32),
        mesh=mesh,
        scratch_types=[
            pltpu.VMEM((rows_t, CHUNK), jnp.int32),
            pltpu.VMEM((rows_t, CHUNK), jnp.float32),
            pltpu.VMEM((CHUNK, 128), jnp.float32),
            pltpu.VMEM((32, 128), jnp.float32),
            pltpu.VMEM_SHARED((NP, 128), jnp.float32),
        ],
    )
    def sc_deg(dstm_hbm, wm_hbm, degp_hbm, dst_v, w_v, rowsb, zbuf, acc_sh):
        c = lax.axis_index("c")
        s = lax.axis_index("s")
        base = pl.multiple_of(c * rows_sc + s * rows_t, 8)
        pltpu.sync_copy(dstm_hbm.at[pl.ds(base, rows_t)], dst_v)
        pltpu.sync_copy(wm_hbm.at[pl.ds(base, rows_t)], w_v)

        zeros16 = jnp.zeros((16,), jnp.float32)

        def zb(i, carry):
            for g in range(8):
                zbuf[i, pl.ds(g * 16, 16)] = zeros16
            return carry

        lax.fori_loop(0, 32, zb, 0)
        fbase = pl.multiple_of(s * nfl, 8)

        def zcopy(i, carry):
            zoff = pl.multiple_of(fbase + i * 32, 8)
            pltpu.sync_copy(zbuf, acc_sh.at[pl.ds(zoff, 32)])
            return carry

        lax.fori_loop(0, nfl // 32, zcopy, 0)
        plsc.subcore_barrier()

        def cbody(r, carry):
            for g in range(CHUNK // 16):
                wvec = w_v[r, pl.ds(g * 16, 16)]
                for l in range(16):
                    wb = jnp.broadcast_to(wvec[l], (16,))
                    for gg in range(8):
                        rowsb[g * 16 + l, pl.ds(gg * 16, 16)] = wb
            pltpu.sync_copy(rowsb, acc_sh.at[dst_v.at[r]], add=True)
            return carry

        lax.fori_loop(0, rows_t, cbody, 0)
        plsc.subcore_barrier()

        def fcopy(i, carry):
            foff = pl.multiple_of(fbase + i * CHUNK, 8)
            pltpu.sync_copy(acc_sh.at[pl.ds(foff, CHUNK)], rowsb)
            pltpu.sync_copy(rowsb, degp_hbm.at[c].at[pl.ds(foff, CHUNK)])
            return carry

        lax.fori_loop(0, nfl // CHUNK, fcopy, 0)

    return sc_deg


def _make_sc_prop(rows):
    rows_t = rows // NS
    half = rows_t // 2
    nflush = NP // NS
    zr = 32
    mesh = plsc.VectorSubcoreMesh(core_axis_name="c", subcore_axis_name="s")

    @functools.partial(
        pl.kernel,
        out_type=jax.ShapeDtypeStruct((4, NP, 128), jnp.float32),
        mesh=mesh,
        scratch_types=[
            pltpu.VMEM((3, CHUNK), jnp.int32),
            pltpu.VMEM((3, CHUNK), jnp.int32),
            pltpu.VMEM((CHUNK, 128), jnp.float32),
            pltpu.VMEM((CHUNK, 128), jnp.float32),
            pltpu.VMEM((zr, 128), jnp.float32),
            pltpu.VMEM_SHARED((NP, 128), jnp.float32),
            pltpu.SemaphoreType.DMA,
            pltpu.SemaphoreType.DMA,
            pltpu.SemaphoreType.DMA,
            pltpu.SemaphoreType.DMA,
            pltpu.SemaphoreType.DMA,
            pltpu.SemaphoreType.DMA,
        ],
    )
    def sc_prop(g_hbm, edm_hbm, acc_hbm, eda, edb, bufa, bufb, zbuf, acc_sh,
                esa, esb, gsa, gsb, ssa, ssb):
        c = lax.axis_index("c")
        s = lax.axis_index("s")
        base = pl.multiple_of(s * rows_t, 8)

        zeros16 = jnp.zeros((16,), jnp.float32)

        def zb(i, carry):
            for g in range(8):
                zbuf[i, pl.ds(g * 16, 16)] = zeros16
            return carry

        lax.fori_loop(0, zr, zb, 0)

        def scale_chunk(ed, buf):
            def gbody(g16, carry):
                wvec = lax.bitcast_convert_type(
                    ed[2, pl.ds(g16 * 16, 16)], jnp.float32)
                for l in range(16):
                    j = g16 * 16 + l
                    wb = jnp.broadcast_to(wvec[l], (16,))
                    for gg in range(8):
                        sl = pl.ds(gg * 16, 16)
                        buf[j, sl] = buf[j, sl] * wb
                return carry

            lax.fori_loop(0, CHUNK // 16, gbody, 0)

        for p in range(2):
            cb = 2 * p + c
            gcb = g_hbm.at[cb]

            fbase = pl.multiple_of(s * nflush, 8)

            def zcopy(i, carry):
                zoff = pl.multiple_of(fbase + i * zr, 8)
                pltpu.sync_copy(zbuf, acc_sh.at[pl.ds(zoff, zr)])
                return carry

            lax.fori_loop(0, nflush // zr, zcopy, 0)
            plsc.subcore_barrier()

            pltpu.async_copy(edm_hbm.at[base], eda, esa)
            pltpu.async_copy(edm_hbm.at[base + 1], edb, esb)
            pltpu.make_async_copy(edm_hbm.at[base], eda, esa).wait()
            pltpu.async_copy(gcb.at[eda.at[0]], bufa, gsa)
            pltpu.make_async_copy(edm_hbm.at[base], edb, esb).wait()
            pltpu.async_copy(gcb.at[edb.at[0]], bufb, gsb)

            def step(i, carry):
                ra = 2 * i
                rb = 2 * i + 1
                pltpu.make_async_copy(gcb.at[eda.at[0]], bufa, gsa).wait()
                scale_chunk(eda, bufa)
                pltpu.async_copy(bufa, acc_sh.at[eda.at[1]], ssa, add=True)
                pltpu.make_async_copy(gcb.at[edb.at[0]], bufb, gsb).wait()
                scale_chunk(edb, bufb)
                pltpu.async_copy(bufb, acc_sh.at[edb.at[1]], ssb, add=True)

                @pl.when(i < half - 1)
                def _():
                    pltpu.make_async_copy(bufa, acc_sh.at[eda.at[1]],
                                          ssa).wait()
                    pltpu.async_copy(edm_hbm.at[base + ra + 2], eda, esa)
                    pltpu.make_async_copy(bufb, acc_sh.at[edb.at[1]],
                                          ssb).wait()
                    pltpu.async_copy(edm_hbm.at[base + rb + 2], edb, esb)
                    pltpu.make_async_copy(edm_hbm.at[base], eda, esa).wait()
                    pltpu.async_copy(gcb.at[eda.at[0]], bufa, gsa)
                    pltpu.make_async_copy(edm_hbm.at[base], edb, esb).wait()
                    pltpu.async_copy(gcb.at[edb.at[0]], bufb, gsb)

                @pl.when(i >= half - 1)
                def _():
                    pltpu.make_async_copy(bufa, acc_sh.at[eda.at[1]],
                                          ssa).wait()
                    pltpu.make_async_copy(bufb, acc_sh.at[edb.at[1]],
                                          ssb).wait()

                return carry

            lax.fori_loop(0, half, step, 0)
            plsc.subcore_barrier()

            def fcopy(i, carry):
                foff = pl.multiple_of(fbase + i * CHUNK, 8)
                pltpu.sync_copy(acc_sh.at[pl.ds(foff, CHUNK)], bufa)
                pltpu.sync_copy(bufa, acc_hbm.at[cb].at[pl.ds(foff, CHUNK)])
                return carry

            lax.fori_loop(0, nflush // CHUNK, fcopy, 0)

    return sc_prop


def _tc1_body(x_ref, w1o_ref, w1g_ref, degp_ref, g1_ref):
    dis = _dis_from_partials(degp_ref[...])[:, None]
    ho = (x_ref[...] @ w1o_ref[...]) * dis
    hg = (x_ref[...] @ w1g_ref[...]) * dis
    g1_ref[0] = ho[:, :128]
    g1_ref[1] = ho[:, 128:]
    g1_ref[2] = hg[:, :128]
    g1_ref[3] = hg[:, 128:]


def _tc2_body(acc_ref, g1_ref, degp_ref, b1o_ref, b1g_ref, w2o_ref, w2g_ref,
              g2_ref):
    dis = _dis_from_partials(degp_ref[...])[:, None]
    m = acc_ref[...] + g1_ref[...]
    mo = jnp.concatenate([m[0], m[1]], axis=1)
    mg = jnp.concatenate([m[2], m[3]], axis=1)
    out1o = jax.nn.relu(mo * dis + b1o_ref[...])
    out1g = jax.nn.relu(mg * dis + b1g_ref[...])
    ho = (out1o @ w2o_ref[...]) * dis
    hg = (out1g @ w2g_ref[...]) * dis
    g2_ref[0] = ho[:, :128]
    g2_ref[1] = ho[:, 128:]
    g2_ref[2] = hg[:, :128]
    g2_ref[3] = hg[:, 128:]


def _tc3_body(acc_ref, g2_ref, degp_ref, b2o_ref, b2g_ref, wlo_ref, wlg_ref,
              blo_ref, blg_ref, x_ref, c_ref, xn_ref, gam_ref):
    dis = _dis_from_partials(degp_ref[...])[:, None]
    m = acc_ref[...] + g2_ref[...]
    out2o = jnp.concatenate([m[0], m[1]], axis=1) * dis + b2o_ref[...]
    out2g = jnp.concatenate([m[2], m[3]], axis=1) * dis + b2g_ref[...]
    z_o = out2o @ wlo_ref[...] + blo_ref[...]
    z_g = out2g @ wlg_ref[...] + blg_ref[...]
    lo = c_ref[0]
    hi = c_ref[1]
    x_sol = lo + (hi - lo) * jax.nn.sigmoid(z_o)
    gamma = jax.nn.sigmoid(z_g)
    xn_ref[...] = x_ref[...] + gamma * (x_sol - x_ref[...])
    gam_ref[...] = gamma


_XB = pl.BlockSpec((ROW_BLK, D), lambda i: (i, 0))
_GB = pl.BlockSpec((4, ROW_BLK, 128), lambda i: (0, i, 0))
_DEGP = pl.BlockSpec((NC, ROW_BLK, 128), lambda i: (0, i, 0))
_W = pl.BlockSpec((D, D), lambda i: (0, 0))
_B = pl.BlockSpec((D,), lambda i: (0,))


def kernel(x, edge_index, edge_weights, constraints, W1_o, W2_o, Wl_o, W1_g,
           W2_g, Wl_g, b1_o, b2_o, bl_o, b1_g, b2_g, bl_g):
    e = edge_weights.shape[0]
    quantum = NC * NS * CHUNK
    e_pad = -(-e // quantum) * quantum
    pad = e_pad - e
    rows = e_pad // CHUNK

    src = jnp.concatenate([edge_index[0], jnp.zeros((pad,), jnp.int32)])
    dst = jnp.concatenate([edge_index[1], jnp.zeros((pad,), jnp.int32)])
    w = jnp.concatenate([edge_weights, jnp.zeros((pad,), jnp.float32)])
    wbits = lax.bitcast_convert_type(w, jnp.int32)
    srcm = src.reshape(rows, CHUNK)
    dstm = dst.reshape(rows, CHUNK)
    wm = w.reshape(rows, CHUNK)
    edm = jnp.stack([srcm, dstm, wbits.reshape(rows, CHUNK)],
                    axis=1)

    degp = _make_sc_deg(rows)(dstm, wm)

    grid = (N // ROW_BLK,)
    g1 = pl.pallas_call(
        _tc1_body,
        grid=grid,
        in_specs=[_XB, _W, _W, _DEGP],
        out_specs=_GB,
        out_shape=jax.ShapeDtypeStruct((4, NP, 128), jnp.float32),
    )(x, W1_o, W1_g, degp)

    sc_prop = _make_sc_prop(rows)
    acc1 = sc_prop(g1, edm)

    g2 = pl.pallas_call(
        _tc2_body,
        grid=grid,
        in_specs=[_GB, _GB, _DEGP, _B, _B, _W, _W],
        out_specs=_GB,
        out_shape=jax.ShapeDtypeStruct((4, NP, 128), jnp.float32),
    )(acc1, g1, degp, b1_o, b1_g, W2_o, W2_g)

    acc2 = sc_prop(g2, edm)

    x_new, gamma = pl.pallas_call(
        _tc3_body,
        grid=grid,
        in_specs=[_GB, _GB, _DEGP, _B, _B, _W, _W, _B, _B, _XB,
                  pl.BlockSpec((2,), lambda i: (0,))],
        out_specs=(_XB, _XB),
        out_shape=(
            jax.ShapeDtypeStruct((N, D), jnp.float32),
            jax.ShapeDtypeStruct((N, D), jnp.float32),
        ),
    )(acc2, g2, degp, b2_o, b2_g, Wl_o, Wl_g, bl_o, bl_g, x, constraints)
    return (x_new, gamma)


# speedup vs baseline: 1.0702x; 1.0702x over previous
import functools

import jax
import jax.numpy as jnp
from jax import lax
from jax.experimental import pallas as pl
from jax.experimental.pallas import tpu as pltpu
from jax.experimental.pallas import tpu_sc as plsc

N = 10000
NP = 10240
D = 256
CHUNK = 128
NC = 2
NS = 16
ROW_BLK = 400


def _dis_from_partials(degp):
    deg = degp[0, :, 0] + degp[1, :, 0] + 1.0
    return jnp.where(deg > 0.0, lax.rsqrt(jnp.maximum(deg, 1e-12)), 0.0)


def _make_sc_deg(rows):
    rows_sc = rows // NC
    rows_t = rows_sc // NS
    nfl = NP // NS
    mesh = plsc.VectorSubcoreMesh(core_axis_name="c", subcore_axis_name="s")

    @functools.partial(
        pl.kernel,
        out_type=jax.ShapeDtypeStruct((NC, NP, 128), jnp.float32),
        mesh=mesh,
        scratch_types=[
            pltpu.VMEM((rows_t, CHUNK), jnp.int32),
            pltpu.VMEM((rows_t, CHUNK), jnp.float32),
            pltpu.VMEM((CHUNK, 128), jnp.float32),
            pltpu.VMEM((32, 128), jnp.float32),
            pltpu.VMEM_SHARED((NP, 128), jnp.float32),
        ],
    )
    def sc_deg(dstm_hbm, wm_hbm, degp_hbm, dst_v, w_v, rowsb, zbuf, acc_sh):
        c = lax.axis_index("c")
        s = lax.axis_index("s")
        base = pl.multiple_of(c * rows_sc + s * rows_t, 8)
        pltpu.sync_copy(dstm_hbm.at[pl.ds(base, rows_t)], dst_v)
        pltpu.sync_copy(wm_hbm.at[pl.ds(base, rows_t)], w_v)

        zeros16 = jnp.zeros((16,), jnp.float32)

        def zb(i, carry):
            for g in range(8):
                zbuf[i, pl.ds(g * 16, 16)] = zeros16
            return carry

        lax.fori_loop(0, 32, zb, 0)
        fbase = pl.multiple_of(s * nfl, 8)

        def zcopy(i, carry):
            zoff = pl.multiple_of(fbase + i * 32, 8)
            pltpu.sync_copy(zbuf, acc_sh.at[pl.ds(zoff, 32)])
            return carry

        lax.fori_loop(0, nfl // 32, zcopy, 0)
        plsc.subcore_barrier()

        def cbody(r, carry):
            for g in range(CHUNK // 16):
                wvec = w_v[r, pl.ds(g * 16, 16)]
                for l in range(16):
                    wb = jnp.broadcast_to(wvec[l], (16,))
                    for gg in range(8):
                        rowsb[g * 16 + l, pl.ds(gg * 16, 16)] = wb
            pltpu.sync_copy(rowsb, acc_sh.at[dst_v.at[r]], add=True)
            return carry

        lax.fori_loop(0, rows_t, cbody, 0)
        plsc.subcore_barrier()

        def fcopy(i, carry):
            foff = pl.multiple_of(fbase + i * CHUNK, 8)
            pltpu.sync_copy(acc_sh.at[pl.ds(foff, CHUNK)], rowsb)
            pltpu.sync_copy(rowsb, degp_hbm.at[c].at[pl.ds(foff, CHUNK)])
            return carry

        lax.fori_loop(0, nfl // CHUNK, fcopy, 0)

    return sc_deg


def _make_sc_prop(rows):
    rows_t = rows // NS
    half = rows_t // 2
    nflush = NP // NS
    zr = 32
    mesh = plsc.VectorSubcoreMesh(core_axis_name="c", subcore_axis_name="s")

    @functools.partial(
        pl.kernel,
        out_type=jax.ShapeDtypeStruct((4, NP, 128), jnp.float32),
        mesh=mesh,
        scratch_types=[
            pltpu.VMEM((3, CHUNK), jnp.int32),
            pltpu.VMEM((3, CHUNK), jnp.int32),
            pltpu.VMEM((CHUNK, 128), jnp.float32),
            pltpu.VMEM((CHUNK, 128), jnp.float32),
            pltpu.VMEM((zr, 128), jnp.float32),
            pltpu.VMEM_SHARED((NP, 128), jnp.float32),
            pltpu.SemaphoreType.DMA,
            pltpu.SemaphoreType.DMA,
            pltpu.SemaphoreType.DMA,
            pltpu.SemaphoreType.DMA,
            pltpu.SemaphoreType.DMA,
            pltpu.SemaphoreType.DMA,
        ],
    )
    def sc_prop(g_hbm, edm_hbm, acc_hbm, eda, edb, bufa, bufb, zbuf, acc_sh,
                esa, esb, gsa, gsb, ssa, ssb):
        c = lax.axis_index("c")
        s = lax.axis_index("s")
        base = pl.multiple_of(s * rows_t, 8)

        zeros16 = jnp.zeros((16,), jnp.float32)

        def zb(i, carry):
            for g in range(8):
                zbuf[i, pl.ds(g * 16, 16)] = zeros16
            return carry

        lax.fori_loop(0, zr, zb, 0)

        def scale_chunk(ed, buf):
            def gbody(g16, carry):
                wvec = lax.bitcast_convert_type(
                    ed[2, pl.ds(g16 * 16, 16)], jnp.float32)
                for l in range(16):
                    j = g16 * 16 + l
                    wb = jnp.broadcast_to(wvec[l], (16,))
                    for gg in range(8):
                        sl = pl.ds(gg * 16, 16)
                        buf[j, sl] = buf[j, sl] * wb
                return carry

            lax.fori_loop(0, CHUNK // 16, gbody, 0)

        for p in range(2):
            cb = 2 * p + c
            gcb = g_hbm.at[cb]

            fbase = pl.multiple_of(s * nflush, 8)

            def zcopy(i, carry):
                zoff = pl.multiple_of(fbase + i * zr, 8)
                pltpu.sync_copy(zbuf, acc_sh.at[pl.ds(zoff, zr)])
                return carry

            lax.fori_loop(0, nflush // zr, zcopy, 0)
            plsc.subcore_barrier()

            pltpu.sync_copy(edm_hbm.at[base], eda)
            pltpu.async_copy(gcb.at[eda.at[0]], bufa, gsa)

            def step(i, carry):
                ra = 2 * i
                rb = 2 * i + 1
                pltpu.sync_copy(edm_hbm.at[base + rb], edb)
                pltpu.async_copy(gcb.at[edb.at[0]], bufb, gsb)
                pltpu.make_async_copy(gcb.at[eda.at[0]], bufa, gsa).wait()
                scale_chunk(eda, bufa)
                pltpu.sync_copy(bufa, acc_sh.at[eda.at[1]], add=True)

                @pl.when(i < half - 1)
                def _():
                    pltpu.sync_copy(edm_hbm.at[base + ra + 2], eda)
                    pltpu.async_copy(gcb.at[eda.at[0]], bufa, gsa)

                pltpu.make_async_copy(gcb.at[edb.at[0]], bufb, gsb).wait()
                scale_chunk(edb, bufb)
                pltpu.sync_copy(bufb, acc_sh.at[edb.at[1]], add=True)
                return carry

            lax.fori_loop(0, half, step, 0)
            plsc.subcore_barrier()

            def fcopy(i, carry):
                foff = pl.multiple_of(fbase + i * CHUNK, 8)
                pltpu.sync_copy(acc_sh.at[pl.ds(foff, CHUNK)], bufa)
                pltpu.sync_copy(bufa, acc_hbm.at[cb].at[pl.ds(foff, CHUNK)])
                return carry

            lax.fori_loop(0, nflush // CHUNK, fcopy, 0)

    return sc_prop


def _tc1_body(x_ref, w1o_ref, w1g_ref, degp_ref, g1_ref):
    dis = _dis_from_partials(degp_ref[...])[:, None]
    ho = (x_ref[...] @ w1o_ref[...]) * dis
    hg = (x_ref[...] @ w1g_ref[...]) * dis
    g1_ref[0] = ho[:, :128]
    g1_ref[1] = ho[:, 128:]
    g1_ref[2] = hg[:, :128]
    g1_ref[3] = hg[:, 128:]


def _tc2_body(acc_ref, g1_ref, degp_ref, b1o_ref, b1g_ref, w2o_ref, w2g_ref,
              g2_ref):
    dis = _dis_from_partials(degp_ref[...])[:, None]
    m = acc_ref[...] + g1_ref[...]
    mo = jnp.concatenate([m[0], m[1]], axis=1)
    mg = jnp.concatenate([m[2], m[3]], axis=1)
    out1o = jax.nn.relu(mo * dis + b1o_ref[...])
    out1g = jax.nn.relu(mg * dis + b1g_ref[...])
    ho = (out1o @ w2o_ref[...]) * dis
    hg = (out1g @ w2g_ref[...]) * dis
    g2_ref[0] = ho[:, :128]
    g2_ref[1] = ho[:, 128:]
    g2_ref[2] = hg[:, :128]
    g2_ref[3] = hg[:, 128:]


def _tc3_body(acc_ref, g2_ref, degp_ref, b2o_ref, b2g_ref, wlo_ref, wlg_ref,
              blo_ref, blg_ref, x_ref, c_ref, xn_ref, gam_ref):
    dis = _dis_from_partials(degp_ref[...])[:, None]
    m = acc_ref[...] + g2_ref[...]
    out2o = jnp.concatenate([m[0], m[1]], axis=1) * dis + b2o_ref[...]
    out2g = jnp.concatenate([m[2], m[3]], axis=1) * dis + b2g_ref[...]
    z_o = out2o @ wlo_ref[...] + blo_ref[...]
    z_g = out2g @ wlg_ref[...] + blg_ref[...]
    lo = c_ref[0]
    hi = c_ref[1]
    x_sol = lo + (hi - lo) * jax.nn.sigmoid(z_o)
    gamma = jax.nn.sigmoid(z_g)
    xn_ref[...] = x_ref[...] + gamma * (x_sol - x_ref[...])
    gam_ref[...] = gamma


_XB = pl.BlockSpec((ROW_BLK, D), lambda i: (i, 0))
_GB = pl.BlockSpec((4, ROW_BLK, 128), lambda i: (0, i, 0))
_DEGP = pl.BlockSpec((NC, ROW_BLK, 128), lambda i: (0, i, 0))
_W = pl.BlockSpec((D, D), lambda i: (0, 0))
_B = pl.BlockSpec((D,), lambda i: (0,))


def kernel(x, edge_index, edge_weights, constraints, W1_o, W2_o, Wl_o, W1_g,
           W2_g, Wl_g, b1_o, b2_o, bl_o, b1_g, b2_g, bl_g):
    e = edge_weights.shape[0]
    quantum = NC * NS * CHUNK
    e_pad = -(-e // quantum) * quantum
    pad = e_pad - e
    rows = e_pad // CHUNK

    src = jnp.concatenate([edge_index[0], jnp.zeros((pad,), jnp.int32)])
    dst = jnp.concatenate([edge_index[1], jnp.zeros((pad,), jnp.int32)])
    w = jnp.concatenate([edge_weights, jnp.zeros((pad,), jnp.float32)])
    wbits = lax.bitcast_convert_type(w, jnp.int32)
    srcm = src.reshape(rows, CHUNK)
    dstm = dst.reshape(rows, CHUNK)
    wm = w.reshape(rows, CHUNK)
    edm = jnp.stack([srcm, dstm, wbits.reshape(rows, CHUNK)],
                    axis=1)

    degp = _make_sc_deg(rows)(dstm, wm)

    grid = (N // ROW_BLK,)
    g1 = pl.pallas_call(
        _tc1_body,
        grid=grid,
        in_specs=[_XB, _W, _W, _DEGP],
        out_specs=_GB,
        out_shape=jax.ShapeDtypeStruct((4, NP, 128), jnp.float32),
    )(x, W1_o, W1_g, degp)

    sc_prop = _make_sc_prop(rows)
    acc1 = sc_prop(g1, edm)

    g2 = pl.pallas_call(
        _tc2_body,
        grid=grid,
        in_specs=[_GB, _GB, _DEGP, _B, _B, _W, _W],
        out_specs=_GB,
        out_shape=jax.ShapeDtypeStruct((4, NP, 128), jnp.float32),
    )(acc1, g1, degp, b1_o, b1_g, W2_o, W2_g)

    acc2 = sc_prop(g2, edm)

    x_new, gamma = pl.pallas_call(
        _tc3_body,
        grid=grid,
        in_specs=[_GB, _GB, _DEGP, _B, _B, _W, _W, _B, _B, _XB,
                  pl.BlockSpec((2,), lambda i: (0,))],
        out_specs=(_XB, _XB),
        out_shape=(
            jax.ShapeDtypeStruct((N, D), jnp.float32),
            jax.ShapeDtypeStruct((N, D), jnp.float32),
        ),
    )(acc2, g2, degp, b2_o, b2_g, Wl_o, Wl_g, bl_o, bl_g, x, constraints)
    return (x_new, gamma)
